# baseline (speedup 1.0000x reference)
import functools

import jax
import jax.numpy as jnp
from jax import lax
from jax.experimental import pallas as pl
from jax.experimental.pallas import tpu as pltpu
from jax.experimental.pallas import tpu_sc as plsc

N = 10000
E = 160000
NUM_NBR = 16
AF = 64
NF = 64
H = 128
NG = 256
EPS = 1e-5
F32 = jnp.float32

_NC = 2
_NS = 16
_NW = _NC * _NS
_CH = 128
E_PAD = 163840
_PT = E_PAD // _NW
_CHUNKS = _PT // _CH
_G = 4
_GR = _G * _CH
_NGRP = _CHUNKS // _G

_BE = 3200
_NBLK = E // _BE
_BN = _BE // NUM_NBR
_BD = 2000
_NDBLK = N // _BD


def _softplus(v):
  return jnp.maximum(v, 0.0) + jnp.log(1.0 + jnp.exp(-jnp.abs(v)))


def _sc_gather2(table, idx2d_a, idx2d_b):
  mesh = plsc.VectorSubcoreMesh(core_axis_name="c", subcore_axis_name="s")
  out_t = jax.ShapeDtypeStruct((E_PAD, 128), F32)

  @functools.partial(
      pl.kernel, mesh=mesh,
      out_type=out_t,
      compiler_params=pltpu.CompilerParams(use_tc_tiling_on_sc=False),
      scratch_types=[
          pltpu.VMEM((_CHUNKS, _CH), jnp.int32),
          pltpu.VMEM((_CHUNKS, _CH), jnp.int32),
          pltpu.VMEM((_GR, 64), F32),
          pltpu.VMEM((_GR, 64), F32),
          pltpu.VMEM_SHARED((N, 64), F32),
          pltpu.SemaphoreType.DMA,
          pltpu.SemaphoreType.DMA,
          pltpu.SemaphoreType.DMA,
          pltpu.SemaphoreType.DMA,
      ],
  )
  def k(hbm_tab, ia, ib, opair, ia_v, ib_v, buf0, buf1, tab, gs0, gs1, ss0, ss1):
    sid = lax.axis_index("s")
    wid = sid * _NC + lax.axis_index("c")
    rows_per = N // _NS
    pltpu.sync_copy(hbm_tab.at[pl.ds(sid * rows_per, rows_per)],
                    tab.at[pl.ds(sid * rows_per, rows_per)])
    pltpu.sync_copy(ia.at[pl.ds(wid * _CHUNKS, _CHUNKS)], ia_v)
    pltpu.sync_copy(ib.at[pl.ds(wid * _CHUNKS, _CHUNKS)], ib_v)
    plsc.subcore_barrier()
    base = wid * _PT
    bufs = (buf0, buf1)
    gsems = (gs0, gs1)
    ssems = (ss0, ss1)

    def run_stream(iv, col0):
      for g in range(2):
        for j in range(_G):
          pltpu.async_copy(tab.at[iv.at[g * _G + j]],
                           bufs[g].at[pl.ds(j * _CH, _CH)], gsems[g])

      def body(go, carry):
        for p in range(2):
          g = go * 2 + p
          buf, gsem, ssem = bufs[p], gsems[p], ssems[p]
          pltpu.make_async_copy(opair.at[pl.ds(0, _GR), pl.ds(0, 64)],
                                buf, gsem).wait()
          dst = opair.at[pl.ds(base + g * _GR, _GR), pl.ds(col0, 64)]
          pltpu.async_copy(buf, dst, ssem)
          pltpu.make_async_copy(buf, dst, ssem).wait()

          @pl.when(g + 2 < _NGRP)
          def _(iv=iv, buf=buf, gsem=gsem, g=g):
            for j in range(_G):
              pltpu.async_copy(tab.at[iv.at[(g + 2) * _G + j]],
                               buf.at[pl.ds(j * _CH, _CH)], gsem)
        return carry

      lax.fori_loop(0, _NGRP // 2, body, 0)

    run_stream(ia_v, 0)
    run_stream(ib_v, 64)

  return k(table, idx2d_a, idx2d_b)


def _embed_h0(x2d, atab_pad):
  def body(x_r, t_r, o_r):
    oh = (x_r[...] == lax.broadcasted_iota(jnp.int32, (_BD, 128), 1)).astype(F32)
    o_r[...] = jnp.dot(oh, t_r[...], preferred_element_type=F32)

  return pl.pallas_call(
      body,
      grid=(_NDBLK,),
      in_specs=[
          pl.BlockSpec((_BD, 1), lambda i: (i, 0)),
          pl.BlockSpec((128, 64), lambda i: (0, 0)),
      ],
      out_specs=pl.BlockSpec((_BD, 64), lambda i: (i, 0)),
      out_shape=jax.ShapeDtypeStruct((N, 64), F32),
  )(x2d, atab_pad)


def _embed_ew(ea2d, btab_pad):
  def body(e_r, t_r, o_r):
    oh = (e_r[...] == lax.broadcasted_iota(jnp.int32, (_BE, 64), 1)).astype(F32)
    o_r[...] = jnp.dot(oh, t_r[...], preferred_element_type=F32)

  return pl.pallas_call(
      body,
      grid=(_NBLK,),
      in_specs=[
          pl.BlockSpec((_BE, 1), lambda i: (i, 0)),
          pl.BlockSpec((64, 64), lambda i: (0, 0)),
      ],
      out_specs=pl.BlockSpec((_BE, 64), lambda i: (i, 0)),
      out_shape=jax.ShapeDtypeStruct((E, 64), F32),
  )(ea2d, btab_pad)


def _zmm(pair_r, ew_r, wni_r, we_r):
  return (jnp.dot(pair_r[...], wni_r[...], preferred_element_type=F32)
          + jnp.dot(ew_r[...], we_r[...], preferred_element_type=F32))


def _edge_layer(pair, ew, Wni, We, gcat, bcat):
  def body(pair_r, ew_r, wni_r, we_r, g_r, b_r, ns_r, ew2_r, acc):
    j = pl.program_id(0)
    i = pl.program_id(1)
    z = _zmm(pair_r, ew_r, wni_r, we_r)

    @pl.when(jnp.logical_and(j == 0, i == 0))
    def _():
      acc[...] = jnp.zeros_like(acc)

    @pl.when(j == 0)
    def _():
      acc[0:1, :] += jnp.sum(z, axis=0, keepdims=True)
      acc[1:2, :] += jnp.sum(z * z, axis=0, keepdims=True)

    @pl.when(j == 1)
    def _():
      mean = acc[0:1, :] * (1.0 / E)
      msq = acc[1:2, :] * (1.0 / E)
      var = msq - mean * mean
      scale = g_r[...] * lax.rsqrt(var + EPS)
      shift = b_r[...] - mean * scale
      zn = z * scale + shift
      gate = 1.0 / (1.0 + jnp.exp(-zn[:, 0:64]))
      core = _softplus(zn[:, 64:128])
      pr = gate * core
      ns_r[...] = jnp.sum(pr.reshape(_BN, NUM_NBR, 64), axis=1)
      ew2_r[...] = _softplus(ew_r[...] + zn[:, 128:192])

  return pl.pallas_call(
      body,
      grid=(2, _NBLK),
      in_specs=[
          pl.BlockSpec((_BE, 128), lambda j, i: (i, 0)),
          pl.BlockSpec((_BE, 64), lambda j, i: (i, 0)),
          pl.BlockSpec((128, 192), lambda j, i: (0, 0)),
          pl.BlockSpec((64, 192), lambda j, i: (0, 0)),
          pl.BlockSpec((1, 192), lambda j, i: (0, 0)),
          pl.BlockSpec((1, 192), lambda j, i: (0, 0)),
      ],
      out_specs=[
          pl.BlockSpec((_BN, 64), lambda j, i: (j * i, 0)),
          pl.BlockSpec((_BE, 64), lambda j, i: (j * i, 0)),
      ],
      out_shape=[
          jax.ShapeDtypeStruct((N, 64), F32),
          jax.ShapeDtypeStruct((E, 64), F32),
      ],
      scratch_shapes=[pltpu.VMEM((8, 192), F32)],
  )(pair, ew, Wni, We, gcat, bcat)


def _node_update(h, ns, g4, be4):
  def body(h_r, ns_r, g_r, b_r, o_r):
    ns = ns_r[...]
    m = jnp.mean(ns, axis=0, keepdims=True)
    v = jnp.mean(ns * ns, axis=0, keepdims=True) - m * m
    bn = (ns - m) * (g_r[...] * lax.rsqrt(v + EPS)) + b_r[...]
    o_r[...] = _softplus(h_r[...] + bn)

  return pl.pallas_call(
      body,
      grid=(1,),
      in_specs=[
          pl.BlockSpec((N, 64), lambda i: (0, 0)),
          pl.BlockSpec((N, 64), lambda i: (0, 0)),
          pl.BlockSpec((1, 64), lambda i: (0, 0)),
          pl.BlockSpec((1, 64), lambda i: (0, 0)),
      ],
      out_specs=pl.BlockSpec((N, 64), lambda i: (0, 0)),
      out_shape=jax.ShapeDtypeStruct((N, 64), F32),
  )(h, ns, g4, be4)


def _head(h0, h1, h2, h3, batT, W1a, W1b, W1c, W1d, b1, W2, b2, Wout, bout):
  def body(h0_r, h1_r, h2_r, h3_r, bt_r, wa, wb, wc, wd, b1_r, w2_r, b2_r,
           wo_r, bo_r, out_r, pool, cnt):
    i = pl.program_id(0)
    z = _softplus(jnp.dot(h0_r[...], wa[...], preferred_element_type=F32)
                  + jnp.dot(h1_r[...], wb[...], preferred_element_type=F32)
                  + jnp.dot(h2_r[...], wc[...], preferred_element_type=F32)
                  + jnp.dot(h3_r[...], wd[...], preferred_element_type=F32)
                  + b1_r[...])
    oh = (bt_r[...] == lax.broadcasted_iota(jnp.int32, (_BD, NG), 1)).astype(F32)

    @pl.when(i == 0)
    def _():
      pool[...] = jnp.zeros_like(pool)
      cnt[...] = jnp.zeros_like(cnt)

    dn = (((0,), (0,)), ((), ()))
    pool[...] += lax.dot_general(oh, z, dn, preferred_element_type=F32)
    cnt[...] += lax.dot_general(oh, jnp.ones((_BD, 1), F32), dn,
                                preferred_element_type=F32)

    @pl.when(i == _NDBLK - 1)
    def _():
      pooled = pool[...] / jnp.maximum(cnt[...], 1.0)
      z2 = _softplus(jnp.dot(pooled, w2_r[...], preferred_element_type=F32)
                     + b2_r[...])
      out_r[...] = jnp.dot(z2, wo_r[...], preferred_element_type=F32) + bo_r[...]

  node_spec = pl.BlockSpec((_BD, 64), lambda i: (i, 0))
  return pl.pallas_call(
      body,
      grid=(_NDBLK,),
      in_specs=[
          node_spec, node_spec, node_spec, node_spec,
          pl.BlockSpec((_BD, 1), lambda i: (i, 0)),
          pl.BlockSpec((64, H), lambda i: (0, 0)),
          pl.BlockSpec((64, H), lambda i: (0, 0)),
          pl.BlockSpec((64, H), lambda i: (0, 0)),
          pl.BlockSpec((64, H), lambda i: (0, 0)),
          pl.BlockSpec((1, H), lambda i: (0, 0)),
          pl.BlockSpec((H, H), lambda i: (0, 0)),
          pl.BlockSpec((1, H), lambda i: (0, 0)),
          pl.BlockSpec((H, 1), lambda i: (0, 0)),
          pl.BlockSpec((1, 1), lambda i: (0, 0)),
      ],
      out_specs=pl.BlockSpec((NG, 1), lambda i: (0, 0)),
      out_shape=jax.ShapeDtypeStruct((NG, 1), F32),
      scratch_shapes=[pltpu.VMEM((NG, H), F32), pltpu.VMEM((NG, 1), F32)],
  )(h0, h1, h2, h3, batT, W1a, W1b, W1c, W1d, b1, W2, b2, Wout, bout)


def kernel(x, edge_index, edge_attr, y, batch, atom_table, bond_table,
           Wc, bc, Wf, bf, Wb, bb, g1, be1, g2, be2, g3, be3, g4, be4,
           W1, b1, W2, b2, Wout, bout):
  del y, bc, bf, bb
  x2d = x.astype(jnp.int32).reshape(N, 1)
  ei = edge_index.astype(jnp.int32)
  ea = edge_attr.astype(jnp.int32)
  pad = E_PAD - E
  idx0 = jnp.pad(ei[0], (0, pad)).reshape(E_PAD // _CH, _CH)
  idx1 = jnp.pad(ei[1], (0, pad)).reshape(E_PAD // _CH, _CH)
  atab = jnp.pad(atom_table.astype(F32), ((0, 128 - atom_table.shape[0]), (0, 0)))
  btab = jnp.pad(bond_table.astype(F32), ((0, 64 - bond_table.shape[0]), (0, 0)))

  h = _embed_h0(x2d, atab)
  ew = _embed_ew(ea.reshape(E, 1), btab)

  hs = [h]
  for i in range(3):
    Wni = jnp.concatenate([Wc[i, 0:128], Wf[i, 0:128], Wb[i, 0:128]], axis=1)
    We_ = jnp.concatenate([Wc[i, 128:192], Wf[i, 128:192], Wb[i, 128:192]], axis=1)
    gcat = jnp.concatenate([g1[i], g2[i], g3[i]]).reshape(1, 192)
    bcat = jnp.concatenate([be1[i], be2[i], be3[i]]).reshape(1, 192)

    pair = _sc_gather2(h, idx1, idx0)
    ns, ew = _edge_layer(pair, ew, Wni, We_, gcat, bcat)
    h = _node_update(h, ns, g4[i].reshape(1, 64), be4[i].reshape(1, 64))
    hs.append(h)

  batT = batch.astype(jnp.int32).reshape(N, 1)
  out = _head(hs[0], hs[1], hs[2], hs[3], batT,
              W1[0:64], W1[64:128], W1[128:192], W1[192:256],
              b1.reshape(1, H), W2, b2.reshape(1, H), Wout, bout.reshape(1, 1))
  return out


# speedup vs baseline: 1.1081x; 1.1081x over previous
import functools

import jax
import jax.numpy as jnp
from jax import lax
from jax.experimental import pallas as pl
from jax.experimental.pallas import tpu as pltpu
from jax.experimental.pallas import tpu_sc as plsc

N = 10000
E = 160000
NUM_NBR = 16
AF = 64
NF = 64
H = 128
NG = 256
EPS = 1e-5
F32 = jnp.float32

_NC = 2
_NS = 16
_NW = _NC * _NS
_CH = 128
E_PAD = 163840
_PT = E_PAD // _NW
_CHUNKS = _PT // _CH
_G = 4
_GR = _G * _CH
_NGRP = _CHUNKS // _G

_BE = 6400
_NBLK = E // _BE
_BN = _BE // NUM_NBR
_BD = 2000
_NDBLK = N // _BD


def _softplus(v):
  return jnp.maximum(v, 0.0) + jnp.log(1.0 + jnp.exp(-jnp.abs(v)))


def _sc_gather2(table, idx2d_a, idx2d_b):
  mesh = plsc.VectorSubcoreMesh(core_axis_name="c", subcore_axis_name="s")
  out_t = jax.ShapeDtypeStruct((E_PAD, 128), F32)

  @functools.partial(
      pl.kernel, mesh=mesh,
      out_type=out_t,
      compiler_params=pltpu.CompilerParams(use_tc_tiling_on_sc=False),
      scratch_types=[
          pltpu.VMEM((_CHUNKS, _CH), jnp.int32),
          pltpu.VMEM((_CHUNKS, _CH), jnp.int32),
          pltpu.VMEM((_GR, 64), F32),
          pltpu.VMEM((_GR, 64), F32),
          pltpu.VMEM_SHARED((N, 64), F32),
          pltpu.SemaphoreType.DMA,
          pltpu.SemaphoreType.DMA,
          pltpu.SemaphoreType.DMA,
          pltpu.SemaphoreType.DMA,
      ],
  )
  def k(hbm_tab, ia, ib, opair, ia_v, ib_v, buf0, buf1, tab, gs0, gs1, ss0, ss1):
    sid = lax.axis_index("s")
    wid = sid * _NC + lax.axis_index("c")
    rows_per = N // _NS
    pltpu.sync_copy(hbm_tab.at[pl.ds(sid * rows_per, rows_per)],
                    tab.at[pl.ds(sid * rows_per, rows_per)])
    pltpu.sync_copy(ia.at[pl.ds(wid * _CHUNKS, _CHUNKS)], ia_v)
    pltpu.sync_copy(ib.at[pl.ds(wid * _CHUNKS, _CHUNKS)], ib_v)
    plsc.subcore_barrier()
    base = wid * _PT
    bufs = (buf0, buf1)
    gsems = (gs0, gs1)
    ssems = (ss0, ss1)

    def run_stream(iv, col0):
      for g in range(2):
        for j in range(_G):
          pltpu.async_copy(tab.at[iv.at[g * _G + j]],
                           bufs[g].at[pl.ds(j * _CH, _CH)], gsems[g])

      def body(go, carry):
        for p in range(2):
          g = go * 2 + p
          buf, gsem, ssem = bufs[p], gsems[p], ssems[p]
          pltpu.make_async_copy(opair.at[pl.ds(0, _GR), pl.ds(0, 64)],
                                buf, gsem).wait()
          dst = opair.at[pl.ds(base + g * _GR, _GR), pl.ds(col0, 64)]
          pltpu.async_copy(buf, dst, ssem)
          pltpu.make_async_copy(buf, dst, ssem).wait()

          @pl.when(g + 2 < _NGRP)
          def _(iv=iv, buf=buf, gsem=gsem, g=g):
            for j in range(_G):
              pltpu.async_copy(tab.at[iv.at[(g + 2) * _G + j]],
                               buf.at[pl.ds(j * _CH, _CH)], gsem)
        return carry

      lax.fori_loop(0, _NGRP // 2, body, 0)

    run_stream(ia_v, 0)
    run_stream(ib_v, 64)

  return k(table, idx2d_a, idx2d_b)


def _embed_h0(x2d, atab_pad):
  def body(x_r, t_r, o_r):
    oh = (x_r[...] == lax.broadcasted_iota(jnp.int32, (_BD, 128), 1)).astype(F32)
    o_r[...] = jnp.dot(oh, t_r[...], preferred_element_type=F32)

  return pl.pallas_call(
      body,
      grid=(_NDBLK,),
      in_specs=[
          pl.BlockSpec((_BD, 1), lambda i: (i, 0)),
          pl.BlockSpec((128, 64), lambda i: (0, 0)),
      ],
      out_specs=pl.BlockSpec((_BD, 64), lambda i: (i, 0)),
      out_shape=jax.ShapeDtypeStruct((N, 64), F32),
  )(x2d, atab_pad)


def _embed_ew(ea2d, btab_pad):
  def body(e_r, t_r, o_r):
    oh = (e_r[...] == lax.broadcasted_iota(jnp.int32, (_BE, 64), 1)).astype(F32)
    o_r[...] = jnp.dot(oh, t_r[...], preferred_element_type=F32)

  return pl.pallas_call(
      body,
      grid=(_NBLK,),
      in_specs=[
          pl.BlockSpec((_BE, 1), lambda i: (i, 0)),
          pl.BlockSpec((64, 64), lambda i: (0, 0)),
      ],
      out_specs=pl.BlockSpec((_BE, 64), lambda i: (i, 0)),
      out_shape=jax.ShapeDtypeStruct((E, 64), F32),
  )(ea2d, btab_pad)


def _zmm(pair_r, ew_r, wni_r, we_r):
  return (jnp.dot(pair_r[...], wni_r[...], preferred_element_type=F32)
          + jnp.dot(ew_r[...], we_r[...], preferred_element_type=F32))


def _edge_specs():
  return [
      pl.BlockSpec((_BE, 128), lambda i: (i, 0)),
      pl.BlockSpec((_BE, 64), lambda i: (i, 0)),
      pl.BlockSpec((128, 192), lambda i: (0, 0)),
      pl.BlockSpec((64, 192), lambda i: (0, 0)),
  ]


def _pass1(pair, ew, Wni, We):
  def body(pair_r, ew_r, wni_r, we_r, out_r, acc):
    i = pl.program_id(0)
    z = _zmm(pair_r, ew_r, wni_r, we_r)

    @pl.when(i == 0)
    def _():
      acc[...] = jnp.zeros_like(acc)

    ones_row = jnp.ones((1, _BE), F32)
    acc[0:1, :] += jnp.dot(ones_row, z, preferred_element_type=F32)
    acc[1:2, :] += jnp.dot(ones_row, z * z, preferred_element_type=F32)

    @pl.when(i == _NBLK - 1)
    def _():
      out_r[...] = acc[...]

  return pl.pallas_call(
      body,
      grid=(_NBLK,),
      in_specs=_edge_specs(),
      out_specs=pl.BlockSpec((8, 192), lambda i: (0, 0)),
      out_shape=jax.ShapeDtypeStruct((8, 192), F32),
      scratch_shapes=[pltpu.VMEM((8, 192), F32)],
  )(pair, ew, Wni, We)


def _pass2(pair, ew, Wni, We, stats, gcat, bcat):
  def body(pair_r, ew_r, wni_r, we_r, st_r, g_r, b_r, ns_r, ew2_r):
    z = _zmm(pair_r, ew_r, wni_r, we_r)
    mean = st_r[0:1, :] * (1.0 / E)
    msq = st_r[1:2, :] * (1.0 / E)
    var = msq - mean * mean
    scale = g_r[...] * lax.rsqrt(var + EPS)
    shift = b_r[...] - mean * scale
    zn = z * scale + shift
    gate = 1.0 / (1.0 + jnp.exp(-zn[:, 0:64]))
    core = _softplus(zn[:, 64:128])
    pr = gate * core
    ns_r[...] = jnp.sum(pr.reshape(_BN, NUM_NBR, 64), axis=1)
    ew2_r[...] = _softplus(ew_r[...] + zn[:, 128:192])

  return pl.pallas_call(
      body,
      grid=(_NBLK,),
      in_specs=_edge_specs() + [
          pl.BlockSpec((8, 192), lambda i: (0, 0)),
          pl.BlockSpec((1, 192), lambda i: (0, 0)),
          pl.BlockSpec((1, 192), lambda i: (0, 0)),
      ],
      out_specs=[
          pl.BlockSpec((_BN, 64), lambda i: (i, 0)),
          pl.BlockSpec((_BE, 64), lambda i: (i, 0)),
      ],
      out_shape=[
          jax.ShapeDtypeStruct((N, 64), F32),
          jax.ShapeDtypeStruct((E, 64), F32),
      ],
  )(pair, ew, Wni, We, stats, gcat, bcat)


def _node_update(h, ns, g4, be4):
  def body(h_r, ns_r, g_r, b_r, o_r):
    ns = ns_r[...]
    m = jnp.mean(ns, axis=0, keepdims=True)
    v = jnp.mean(ns * ns, axis=0, keepdims=True) - m * m
    bn = (ns - m) * (g_r[...] * lax.rsqrt(v + EPS)) + b_r[...]
    o_r[...] = _softplus(h_r[...] + bn)

  return pl.pallas_call(
      body,
      grid=(1,),
      in_specs=[
          pl.BlockSpec((N, 64), lambda i: (0, 0)),
          pl.BlockSpec((N, 64), lambda i: (0, 0)),
          pl.BlockSpec((1, 64), lambda i: (0, 0)),
          pl.BlockSpec((1, 64), lambda i: (0, 0)),
      ],
      out_specs=pl.BlockSpec((N, 64), lambda i: (0, 0)),
      out_shape=jax.ShapeDtypeStruct((N, 64), F32),
  )(h, ns, g4, be4)


def _head(h0, h1, h2, h3, batT, W1a, W1b, W1c, W1d, b1, W2, b2, Wout, bout):
  def body(h0_r, h1_r, h2_r, h3_r, bt_r, wa, wb, wc, wd, b1_r, w2_r, b2_r,
           wo_r, bo_r, out_r, pool, cnt):
    i = pl.program_id(0)
    z = _softplus(jnp.dot(h0_r[...], wa[...], preferred_element_type=F32)
                  + jnp.dot(h1_r[...], wb[...], preferred_element_type=F32)
                  + jnp.dot(h2_r[...], wc[...], preferred_element_type=F32)
                  + jnp.dot(h3_r[...], wd[...], preferred_element_type=F32)
                  + b1_r[...])
    oh = (bt_r[...] == lax.broadcasted_iota(jnp.int32, (_BD, NG), 1)).astype(F32)

    @pl.when(i == 0)
    def _():
      pool[...] = jnp.zeros_like(pool)
      cnt[...] = jnp.zeros_like(cnt)

    dn = (((0,), (0,)), ((), ()))
    pool[...] += lax.dot_general(oh, z, dn, preferred_element_type=F32)
    cnt[...] += lax.dot_general(oh, jnp.ones((_BD, 1), F32), dn,
                                preferred_element_type=F32)

    @pl.when(i == _NDBLK - 1)
    def _():
      pooled = pool[...] / jnp.maximum(cnt[...], 1.0)
      z2 = _softplus(jnp.dot(pooled, w2_r[...], preferred_element_type=F32)
                     + b2_r[...])
      out_r[...] = jnp.dot(z2, wo_r[...], preferred_element_type=F32) + bo_r[...]

  node_spec = pl.BlockSpec((_BD, 64), lambda i: (i, 0))
  return pl.pallas_call(
      body,
      grid=(_NDBLK,),
      in_specs=[
          node_spec, node_spec, node_spec, node_spec,
          pl.BlockSpec((_BD, 1), lambda i: (i, 0)),
          pl.BlockSpec((64, H), lambda i: (0, 0)),
          pl.BlockSpec((64, H), lambda i: (0, 0)),
          pl.BlockSpec((64, H), lambda i: (0, 0)),
          pl.BlockSpec((64, H), lambda i: (0, 0)),
          pl.BlockSpec((1, H), lambda i: (0, 0)),
          pl.BlockSpec((H, H), lambda i: (0, 0)),
          pl.BlockSpec((1, H), lambda i: (0, 0)),
          pl.BlockSpec((H, 1), lambda i: (0, 0)),
          pl.BlockSpec((1, 1), lambda i: (0, 0)),
      ],
      out_specs=pl.BlockSpec((NG, 1), lambda i: (0, 0)),
      out_shape=jax.ShapeDtypeStruct((NG, 1), F32),
      scratch_shapes=[pltpu.VMEM((NG, H), F32), pltpu.VMEM((NG, 1), F32)],
  )(h0, h1, h2, h3, batT, W1a, W1b, W1c, W1d, b1, W2, b2, Wout, bout)


def kernel(x, edge_index, edge_attr, y, batch, atom_table, bond_table,
           Wc, bc, Wf, bf, Wb, bb, g1, be1, g2, be2, g3, be3, g4, be4,
           W1, b1, W2, b2, Wout, bout):
  del y, bc, bf, bb
  x2d = x.astype(jnp.int32).reshape(N, 1)
  ei = edge_index.astype(jnp.int32)
  ea = edge_attr.astype(jnp.int32)
  pad = E_PAD - E
  idx0 = jnp.pad(ei[0], (0, pad)).reshape(E_PAD // _CH, _CH)
  idx1 = jnp.pad(ei[1], (0, pad)).reshape(E_PAD // _CH, _CH)
  atab = jnp.pad(atom_table.astype(F32), ((0, 128 - atom_table.shape[0]), (0, 0)))
  btab = jnp.pad(bond_table.astype(F32), ((0, 64 - bond_table.shape[0]), (0, 0)))

  h = _embed_h0(x2d, atab)
  ew = _embed_ew(ea.reshape(E, 1), btab)

  hs = [h]
  for i in range(3):
    Wni = jnp.concatenate([Wc[i, 0:128], Wf[i, 0:128], Wb[i, 0:128]], axis=1)
    We_ = jnp.concatenate([Wc[i, 128:192], Wf[i, 128:192], Wb[i, 128:192]], axis=1)
    gcat = jnp.concatenate([g1[i], g2[i], g3[i]]).reshape(1, 192)
    bcat = jnp.concatenate([be1[i], be2[i], be3[i]]).reshape(1, 192)

    pair = _sc_gather2(h, idx1, idx0)
    stats = _pass1(pair, ew, Wni, We_)
    ns, ew = _pass2(pair, ew, Wni, We_, stats, gcat, bcat)
    h = _node_update(h, ns, g4[i].reshape(1, 64), be4[i].reshape(1, 64))
    hs.append(h)

  batT = batch.astype(jnp.int32).reshape(N, 1)
  out = _head(hs[0], hs[1], hs[2], hs[3], batT,
              W1[0:64], W1[64:128], W1[128:192], W1[192:256],
              b1.reshape(1, H), W2, b2.reshape(1, H), Wout, bout.reshape(1, 1))
  return out


# speedup vs baseline: 1.2567x; 1.1342x over previous
import functools

import jax
import jax.numpy as jnp
from jax import lax
from jax.experimental import pallas as pl
from jax.experimental.pallas import tpu as pltpu
from jax.experimental.pallas import tpu_sc as plsc

N = 10000
E = 160000
NUM_NBR = 16
AF = 64
NF = 64
H = 128
NG = 256
EPS = 1e-5
F32 = jnp.float32

_NC = 2
_NS = 16
_NW = _NC * _NS
_CH = 128
E_PAD = 163840
_PT = E_PAD // _NW
_CHUNKS = _PT // _CH
_G = 4
_GR = _G * _CH
_NGRP = _CHUNKS // _G

_BE = 6400
_NBLK = E // _BE
_BN = _BE // NUM_NBR
_BD = 2000
_NDBLK = N // _BD


def _softplus(v):
  return jnp.maximum(v, 0.0) + jnp.log(1.0 + jnp.exp(-jnp.abs(v)))


def _sc_gather2(table, idx2d_a, idx2d_b):
  mesh = plsc.VectorSubcoreMesh(core_axis_name="c", subcore_axis_name="s")
  out_t = jax.ShapeDtypeStruct((E_PAD, 128), F32)

  @functools.partial(
      pl.kernel, mesh=mesh,
      out_type=out_t,
      compiler_params=pltpu.CompilerParams(use_tc_tiling_on_sc=False),
      scratch_types=[
          pltpu.VMEM((_CHUNKS, _CH), jnp.int32),
          pltpu.VMEM((_CHUNKS, _CH), jnp.int32),
          pltpu.VMEM((_GR, 64), F32),
          pltpu.VMEM((_GR, 64), F32),
          pltpu.VMEM_SHARED((N, 64), F32),
          pltpu.SemaphoreType.DMA,
          pltpu.SemaphoreType.DMA,
          pltpu.SemaphoreType.DMA,
          pltpu.SemaphoreType.DMA,
      ],
  )
  def k(hbm_tab, ia, ib, opair, ia_v, ib_v, buf0, buf1, tab, gs0, gs1, ss0, ss1):
    sid = lax.axis_index("s")
    wid = sid * _NC + lax.axis_index("c")
    rows_per = N // _NS
    pltpu.sync_copy(hbm_tab.at[pl.ds(sid * rows_per, rows_per)],
                    tab.at[pl.ds(sid * rows_per, rows_per)])
    pltpu.sync_copy(ia.at[pl.ds(wid * _CHUNKS, _CHUNKS)], ia_v)
    pltpu.sync_copy(ib.at[pl.ds(wid * _CHUNKS, _CHUNKS)], ib_v)
    plsc.subcore_barrier()
    base = wid * _PT
    bufs = (buf0, buf1)
    gsems = (gs0, gs1)
    ssems = (ss0, ss1)

    def run_stream(iv, col0):
      for g in range(2):
        for j in range(_G):
          pltpu.async_copy(tab.at[iv.at[g * _G + j]],
                           bufs[g].at[pl.ds(j * _CH, _CH)], gsems[g])

      def body(go, carry):
        for p in range(2):
          g = go * 2 + p
          buf, gsem, ssem = bufs[p], gsems[p], ssems[p]
          pltpu.make_async_copy(opair.at[pl.ds(0, _GR), pl.ds(0, 64)],
                                buf, gsem).wait()
          dst = opair.at[pl.ds(base + g * _GR, _GR), pl.ds(col0, 64)]
          pltpu.async_copy(buf, dst, ssem)
          pltpu.make_async_copy(buf, dst, ssem).wait()

          @pl.when(g + 2 < _NGRP)
          def _(iv=iv, buf=buf, gsem=gsem, g=g):
            for j in range(_G):
              pltpu.async_copy(tab.at[iv.at[(g + 2) * _G + j]],
                               buf.at[pl.ds(j * _CH, _CH)], gsem)
        return carry

      lax.fori_loop(0, _NGRP // 2, body, 0)

    run_stream(ia_v, 0)
    run_stream(ib_v, 64)

  return k(table, idx2d_a, idx2d_b)


def _embed_h0(x2d, atab_pad):
  def body(x_r, t_r, o_r):
    oh = (x_r[...] == lax.broadcasted_iota(jnp.int32, (_BD, 128), 1)).astype(F32)
    o_r[...] = jnp.dot(oh, t_r[...], preferred_element_type=F32)

  return pl.pallas_call(
      body,
      grid=(_NDBLK,),
      in_specs=[
          pl.BlockSpec((_BD, 1), lambda i: (i, 0)),
          pl.BlockSpec((128, 64), lambda i: (0, 0)),
      ],
      out_specs=pl.BlockSpec((_BD, 64), lambda i: (i, 0)),
      out_shape=jax.ShapeDtypeStruct((N, 64), F32),
  )(x2d, atab_pad)


def _embed_ew(ea2d, btab_pad):
  def body(e_r, t_r, o_r):
    oh = (e_r[...] == lax.broadcasted_iota(jnp.int32, (_BE, 64), 1)).astype(F32)
    o_r[...] = jnp.dot(oh, t_r[...], preferred_element_type=F32)

  return pl.pallas_call(
      body,
      grid=(_NBLK,),
      in_specs=[
          pl.BlockSpec((_BE, 1), lambda i: (i, 0)),
          pl.BlockSpec((64, 64), lambda i: (0, 0)),
      ],
      out_specs=pl.BlockSpec((_BE, 64), lambda i: (i, 0)),
      out_shape=jax.ShapeDtypeStruct((E, 64), F32),
  )(ea2d, btab_pad)


def _zmm(pair_r, ew_r, wni_r, we_r):
  return (jnp.dot(pair_r[...], wni_r[...], preferred_element_type=F32)
          + jnp.dot(ew_r[...], we_r[...], preferred_element_type=F32))


def _edge_specs():
  return [
      pl.BlockSpec((_BE, 128), lambda i: (i, 0)),
      pl.BlockSpec((_BE, 64), lambda i: (i, 0)),
      pl.BlockSpec((128, 192), lambda i: (0, 0)),
      pl.BlockSpec((64, 192), lambda i: (0, 0)),
  ]


def _pass1(pair, ew, Wni, We):
  def body(pair_r, ew_r, wni_r, we_r, out_r, acc):
    i = pl.program_id(0)
    z = _zmm(pair_r, ew_r, wni_r, we_r)

    @pl.when(i == 0)
    def _():
      acc[...] = jnp.zeros_like(acc)

    acc[0:1, :] += jnp.sum(z, axis=0, keepdims=True)
    acc[1:2, :] += jnp.sum(z * z, axis=0, keepdims=True)

    @pl.when(i == _NBLK - 1)
    def _():
      out_r[...] = acc[...]

  return pl.pallas_call(
      body,
      grid=(_NBLK,),
      in_specs=_edge_specs(),
      out_specs=pl.BlockSpec((8, 192), lambda i: (0, 0)),
      out_shape=jax.ShapeDtypeStruct((8, 192), F32),
      scratch_shapes=[pltpu.VMEM((8, 192), F32)],
  )(pair, ew, Wni, We)


def _pass2(pair, ew, Wni, We, stats, gcat, bcat):
  def body(pair_r, ew_r, wni_r, we_r, st_r, g_r, b_r, ns_r, ew2_r):
    z = _zmm(pair_r, ew_r, wni_r, we_r)
    mean = st_r[0:1, :] * (1.0 / E)
    msq = st_r[1:2, :] * (1.0 / E)
    var = msq - mean * mean
    scale = g_r[...] * lax.rsqrt(var + EPS)
    shift = b_r[...] - mean * scale
    zn = z * scale + shift
    gate = 1.0 / (1.0 + jnp.exp(-zn[:, 0:64]))
    core = _softplus(zn[:, 64:128])
    pr = gate * core
    ns_r[...] = jnp.sum(pr.reshape(_BN, NUM_NBR, 64), axis=1)
    ew2_r[...] = _softplus(ew_r[...] + zn[:, 128:192])

  return pl.pallas_call(
      body,
      grid=(_NBLK,),
      in_specs=_edge_specs() + [
          pl.BlockSpec((8, 192), lambda i: (0, 0)),
          pl.BlockSpec((1, 192), lambda i: (0, 0)),
          pl.BlockSpec((1, 192), lambda i: (0, 0)),
      ],
      out_specs=[
          pl.BlockSpec((_BN, 64), lambda i: (i, 0)),
          pl.BlockSpec((_BE, 64), lambda i: (i, 0)),
      ],
      out_shape=[
          jax.ShapeDtypeStruct((N, 64), F32),
          jax.ShapeDtypeStruct((E, 64), F32),
      ],
  )(pair, ew, Wni, We, stats, gcat, bcat)


def _node_update(h, ns, g4, be4):
  def body(h_r, ns_r, g_r, b_r, o_r):
    ns = ns_r[...]
    m = jnp.mean(ns, axis=0, keepdims=True)
    v = jnp.mean(ns * ns, axis=0, keepdims=True) - m * m
    bn = (ns - m) * (g_r[...] * lax.rsqrt(v + EPS)) + b_r[...]
    o_r[...] = _softplus(h_r[...] + bn)

  return pl.pallas_call(
      body,
      grid=(1,),
      in_specs=[
          pl.BlockSpec((N, 64), lambda i: (0, 0)),
          pl.BlockSpec((N, 64), lambda i: (0, 0)),
          pl.BlockSpec((1, 64), lambda i: (0, 0)),
          pl.BlockSpec((1, 64), lambda i: (0, 0)),
      ],
      out_specs=pl.BlockSpec((N, 64), lambda i: (0, 0)),
      out_shape=jax.ShapeDtypeStruct((N, 64), F32),
  )(h, ns, g4, be4)


def _head(h0, h1, h2, h3, batT, W1a, W1b, W1c, W1d, b1, W2, b2, Wout, bout):
  def body(h0_r, h1_r, h2_r, h3_r, bt_r, wa, wb, wc, wd, b1_r, w2_r, b2_r,
           wo_r, bo_r, out_r, pool, cnt):
    i = pl.program_id(0)
    z = _softplus(jnp.dot(h0_r[...], wa[...], preferred_element_type=F32)
                  + jnp.dot(h1_r[...], wb[...], preferred_element_type=F32)
                  + jnp.dot(h2_r[...], wc[...], preferred_element_type=F32)
                  + jnp.dot(h3_r[...], wd[...], preferred_element_type=F32)
                  + b1_r[...])
    oh = (bt_r[...] == lax.broadcasted_iota(jnp.int32, (_BD, NG), 1)).astype(F32)

    @pl.when(i == 0)
    def _():
      pool[...] = jnp.zeros_like(pool)
      cnt[...] = jnp.zeros_like(cnt)

    dn = (((0,), (0,)), ((), ()))
    pool[...] += lax.dot_general(oh, z, dn, preferred_element_type=F32)
    cnt[...] += lax.dot_general(oh, jnp.ones((_BD, 1), F32), dn,
                                preferred_element_type=F32)

    @pl.when(i == _NDBLK - 1)
    def _():
      pooled = pool[...] / jnp.maximum(cnt[...], 1.0)
      z2 = _softplus(jnp.dot(pooled, w2_r[...], preferred_element_type=F32)
                     + b2_r[...])
      out_r[...] = jnp.dot(z2, wo_r[...], preferred_element_type=F32) + bo_r[...]

  node_spec = pl.BlockSpec((_BD, 64), lambda i: (i, 0))
  return pl.pallas_call(
      body,
      grid=(_NDBLK,),
      in_specs=[
          node_spec, node_spec, node_spec, node_spec,
          pl.BlockSpec((_BD, 1), lambda i: (i, 0)),
          pl.BlockSpec((64, H), lambda i: (0, 0)),
          pl.BlockSpec((64, H), lambda i: (0, 0)),
          pl.BlockSpec((64, H), lambda i: (0, 0)),
          pl.BlockSpec((64, H), lambda i: (0, 0)),
          pl.BlockSpec((1, H), lambda i: (0, 0)),
          pl.BlockSpec((H, H), lambda i: (0, 0)),
          pl.BlockSpec((1, H), lambda i: (0, 0)),
          pl.BlockSpec((H, 1), lambda i: (0, 0)),
          pl.BlockSpec((1, 1), lambda i: (0, 0)),
      ],
      out_specs=pl.BlockSpec((NG, 1), lambda i: (0, 0)),
      out_shape=jax.ShapeDtypeStruct((NG, 1), F32),
      scratch_shapes=[pltpu.VMEM((NG, H), F32), pltpu.VMEM((NG, 1), F32)],
  )(h0, h1, h2, h3, batT, W1a, W1b, W1c, W1d, b1, W2, b2, Wout, bout)


def kernel(x, edge_index, edge_attr, y, batch, atom_table, bond_table,
           Wc, bc, Wf, bf, Wb, bb, g1, be1, g2, be2, g3, be3, g4, be4,
           W1, b1, W2, b2, Wout, bout):
  del y, bc, bf, bb
  x2d = x.astype(jnp.int32).reshape(N, 1)
  ei = edge_index.astype(jnp.int32)
  ea = edge_attr.astype(jnp.int32)
  pad = E_PAD - E
  idx0 = jnp.pad(ei[0], (0, pad)).reshape(E_PAD // _CH, _CH)
  idx1 = jnp.pad(ei[1], (0, pad)).reshape(E_PAD // _CH, _CH)
  atab = jnp.pad(atom_table.astype(F32), ((0, 128 - atom_table.shape[0]), (0, 0)))
  btab = jnp.pad(bond_table.astype(F32), ((0, 64 - bond_table.shape[0]), (0, 0)))

  h = _embed_h0(x2d, atab)
  ew = _embed_ew(ea.reshape(E, 1), btab)

  hs = [h]
  for i in range(3):
    Wni = jnp.concatenate([Wc[i, 0:128], Wf[i, 0:128], Wb[i, 0:128]], axis=1)
    We_ = jnp.concatenate([Wc[i, 128:192], Wf[i, 128:192], Wb[i, 128:192]], axis=1)
    gcat = jnp.concatenate([g1[i], g2[i], g3[i]]).reshape(1, 192)
    bcat = jnp.concatenate([be1[i], be2[i], be3[i]]).reshape(1, 192)

    pair = _sc_gather2(h, idx1, idx0)
    stats = _pass1(pair, ew, Wni, We_)
    ns, ew = _pass2(pair, ew, Wni, We_, stats, gcat, bcat)
    h = _node_update(h, ns, g4[i].reshape(1, 64), be4[i].reshape(1, 64))
    hs.append(h)

  batT = batch.astype(jnp.int32).reshape(N, 1)
  out = _head(hs[0], hs[1], hs[2], hs[3], batT,
              W1[0:64], W1[64:128], W1[128:192], W1[192:256],
              b1.reshape(1, H), W2, b2.reshape(1, H), Wout, bout.reshape(1, 1))
  return out
